# Initial kernel scaffold; baseline (speedup 1.0000x reference)
import jax
import jax.numpy as jnp
from jax.experimental import pallas as pl


def kernel(x, edge_index, edge_attr, y, params):
    raise NotImplementedError("write your pallas kernel here")


# speedup vs baseline: 1.9176x; 1.9176x over previous
import functools

import jax
import jax.numpy as jnp
from jax import lax
from jax.experimental import pallas as pl
from jax.experimental.pallas import tpu as pltpu
from jax.experimental.pallas import tpu_sc as plsc

N = 10000
E = 320000
H = 128
D_EDGE = 16
L = 3

NC = 2
NS = 16
NW = NC * NS
GROUP = 128
GPW = 80
EW = GROUP * GPW
EPAD = EW * NW
NPAD = 10112
RPW = NPAD // NS
CG = 16

BN = 2000
BE = 4096


def _sc_body(h_hbm, emb_hbm, src_hbm, dst_hbm, zero_hbm, out_hbm,
             srcbuf, dstbuf, gbuf, mbuf, acc, sem):
    c = lax.axis_index("c")
    s = lax.axis_index("s")
    wid = c * NS + s

    pltpu.sync_copy(zero_hbm, acc.at[pl.ds(s * RPW, RPW)])

    plsc.subcore_barrier()

    def chunk_step(cc, carry0):
        pltpu.sync_copy(src_hbm.at[pl.ds(wid * GPW + cc * CG, CG)], srcbuf)
        pltpu.sync_copy(dst_hbm.at[pl.ds(wid * GPW + cc * CG, CG)], dstbuf)

        def group_step(j, carry):
            ebase = wid * EW + (cc * CG + j) * GROUP
            pltpu.sync_copy(emb_hbm.at[pl.ds(ebase, GROUP)], mbuf)
            pltpu.async_copy(h_hbm.at[srcbuf.at[j]], gbuf, sem).wait()

            def row_step(i, carry2):
                for k in range(H // 16):
                    sl = pl.ds(k * 16, 16)
                    mbuf[i, sl] = jnp.maximum(mbuf[i, sl] + gbuf[i, sl], 0.0)
                return carry2

            lax.fori_loop(0, GROUP, row_step, 0)
            pltpu.sync_copy(mbuf, acc.at[dstbuf.at[j]], add=True)
            return carry

        lax.fori_loop(0, CG, group_step, 0)
        return carry0

    lax.fori_loop(0, GPW // CG, chunk_step, 0)
    plsc.subcore_barrier()
    pltpu.sync_copy(acc.at[pl.ds(s * RPW, RPW)],
                    out_hbm.at[c, pl.ds(s * RPW, RPW)])


_sc_gather_scatter = functools.partial(
    pl.kernel,
    out_type=jax.ShapeDtypeStruct((NC, NPAD, H), jnp.float32),
    mesh=plsc.VectorSubcoreMesh(
        core_axis_name="c", subcore_axis_name="s",
        num_cores=NC, num_subcores=NS),
    scratch_types=[
        pltpu.VMEM((CG, GROUP), jnp.int32),
        pltpu.VMEM((CG, GROUP), jnp.int32),
        pltpu.VMEM((GROUP, H), jnp.float32),
        pltpu.VMEM((GROUP, H), jnp.float32),
        pltpu.VMEM_SHARED((NPAD, H), jnp.float32),
        pltpu.SemaphoreType.DMA,
    ],
)(_sc_body)


def _enc_body(x_ref, w_ref, b_ref, o_ref):
    o_ref[...] = (
        jnp.dot(x_ref[...], w_ref[...], preferred_element_type=jnp.float32)
        + b_ref[...])


_encoder = pl.pallas_call(
    _enc_body,
    grid=(N // BN,),
    in_specs=[
        pl.BlockSpec((BN, 128), lambda i: (i, 0)),
        pl.BlockSpec((128, H), lambda i: (0, 0)),
        pl.BlockSpec((1, H), lambda i: (0, 0)),
    ],
    out_specs=pl.BlockSpec((BN, H), lambda i: (i, 0)),
    out_shape=jax.ShapeDtypeStruct((N, H), jnp.float32),
)


def _edge_body(a_ref, w_ref, b_ref, o_ref):
    o_ref[...] = (
        jnp.dot(a_ref[...], w_ref[...], preferred_element_type=jnp.float32)
        + b_ref[...])


_edge_embed = pl.pallas_call(
    _edge_body,
    grid=(EPAD // BE,),
    in_specs=[
        pl.BlockSpec((BE, D_EDGE), lambda i: (i, 0)),
        pl.BlockSpec((D_EDGE, H), lambda i: (0, 0)),
        pl.BlockSpec((1, H), lambda i: (0, 0)),
    ],
    out_specs=pl.BlockSpec((BE, H), lambda i: (i, 0)),
    out_shape=jax.ShapeDtypeStruct((EPAD, H), jnp.float32),
)


def _node_body(eps_ref, h_ref, a_ref, w1_ref, b1_ref, w2_ref, b2_ref, o_ref):
    z = h_ref[...] * eps_ref[0, 0] + a_ref[0] + a_ref[1]
    z = jnp.maximum(
        jnp.dot(z, w1_ref[...], preferred_element_type=jnp.float32)
        + b1_ref[...], 0.0)
    z = (jnp.dot(z, w2_ref[...], preferred_element_type=jnp.float32)
         + b2_ref[...])
    o_ref[...] = jnp.maximum(z, 0.0)


_node_update = pl.pallas_call(
    _node_body,
    grid=(N // BN,),
    in_specs=[
        pl.BlockSpec(memory_space=pltpu.SMEM),
        pl.BlockSpec((BN, H), lambda i: (i, 0)),
        pl.BlockSpec((NC, BN, H), lambda i: (0, i, 0)),
        pl.BlockSpec((H, H), lambda i: (0, 0)),
        pl.BlockSpec((1, H), lambda i: (0, 0)),
        pl.BlockSpec((H, H), lambda i: (0, 0)),
        pl.BlockSpec((1, H), lambda i: (0, 0)),
    ],
    out_specs=pl.BlockSpec((BN, H), lambda i: (i, 0)),
    out_shape=jax.ShapeDtypeStruct((N, H), jnp.float32),
)


def _head_body(h_ref, w0_ref, b0_ref, w1_ref, b1_ref, w2_ref, b2_ref, o_ref):
    o = jnp.maximum(
        jnp.dot(h_ref[...], w0_ref[...], preferred_element_type=jnp.float32)
        + b0_ref[...], 0.0)
    o = jnp.maximum(
        jnp.dot(o, w1_ref[...], preferred_element_type=jnp.float32)
        + b1_ref[...], 0.0)
    o_ref[...] = (
        jnp.dot(o, w2_ref[...], preferred_element_type=jnp.float32)
        + b2_ref[...])


_head = pl.pallas_call(
    _head_body,
    grid=(N // BN,),
    in_specs=[
        pl.BlockSpec((BN, H), lambda i: (i, 0)),
        pl.BlockSpec((H, H), lambda i: (0, 0)),
        pl.BlockSpec((1, H), lambda i: (0, 0)),
        pl.BlockSpec((H, H), lambda i: (0, 0)),
        pl.BlockSpec((1, H), lambda i: (0, 0)),
        pl.BlockSpec((H, H), lambda i: (0, 0)),
        pl.BlockSpec((1, H), lambda i: (0, 0)),
    ],
    out_specs=pl.BlockSpec((BN, H), lambda i: (i, 0)),
    out_shape=jax.ShapeDtypeStruct((N, H), jnp.float32),
)


def kernel(x, edge_index, edge_attr, y, params):
    p = params
    pad = EPAD - E
    src2d = jnp.concatenate(
        [edge_index[0], jnp.zeros((pad,), jnp.int32)]).reshape(EPAD // GROUP, GROUP)
    dst2d = jnp.concatenate(
        [edge_index[1], jnp.full((pad,), N, jnp.int32)]).reshape(EPAD // GROUP, GROUP)
    ea_pad = jnp.concatenate(
        [edge_attr, jnp.zeros((pad, D_EDGE), jnp.float32)], axis=0)
    zero_rows = jnp.zeros((RPW, H), jnp.float32)

    h = _encoder(x, p['enc_Wn'], p['enc_bn'].reshape(1, H))
    for l in range(L):
        wc = p['enc_We'] @ p[f'l{l}_elin_W']
        bc = p['enc_be'] @ p[f'l{l}_elin_W'] + p[f'l{l}_elin_b']
        emb = _edge_embed(ea_pad, wc, bc.reshape(1, H))
        agg2 = _sc_gather_scatter(h, emb, src2d, dst2d, zero_rows)
        g = p[f'l{l}_bn_g']
        w2 = p[f'l{l}_W2'] * g[None, :]
        b2 = p[f'l{l}_b2'] * g + p[f'l{l}_bn_b']
        epsm = (1.0 + p[f'l{l}_eps']).reshape(1, 1)
        h = _node_update(epsm, h, agg2, p[f'l{l}_W1'],
                         p[f'l{l}_b1'].reshape(1, H), w2, b2.reshape(1, H))

    w2p = jnp.pad(p['head_W2'], ((0, 0), (0, 127)))
    b2p = jnp.pad(p['head_b2'], (0, 127)).reshape(1, 128)
    o = _head(h, p['head_W0'], p['head_b0'].reshape(1, H),
              p['head_W1'], p['head_b1'].reshape(1, H), w2p, b2p)
    pred = o[:, :1]

    true_class = jnp.full((N,), -1, jnp.int32)
    true_label = jnp.where(y != -1.0, y, -1.0)
    return (pred, true_class, true_label)


# speedup vs baseline: 2.4292x; 1.2668x over previous
import functools

import jax
import jax.numpy as jnp
from jax import lax
from jax.experimental import pallas as pl
from jax.experimental.pallas import tpu as pltpu
from jax.experimental.pallas import tpu_sc as plsc

N = 10000
E = 320000
H = 128
D_EDGE = 16
L = 3

NC = 2
NS = 16
NW = NC * NS
GROUP = 64
GPW = 160
EW = GROUP * GPW
EPAD = EW * NW
NPAD = 10112
RPW = NPAD // NS
CG = 40

BN = 2000
BE = 4096


def _sc_body(h_hbm, emb_hbm, src_hbm, dst_hbm, zero_hbm, out_hbm,
             srcbuf, dstbuf, gbuf0, mbuf0, gbuf1, mbuf1, acc, sem0, sem1):
    c = lax.axis_index("c")
    s = lax.axis_index("s")
    wid = c * NS + s

    pltpu.sync_copy(zero_hbm, acc.at[pl.ds(s * RPW, RPW)])

    plsc.subcore_barrier()

    def start(cc, j, gbuf, mbuf, sem):
        ebase = wid * EW + cc * CG * GROUP + j * GROUP
        pltpu.async_copy(emb_hbm.at[pl.ds(ebase, GROUP)], mbuf, sem)
        pltpu.async_copy(h_hbm.at[srcbuf.at[j]], gbuf, sem)

    def drain(gbuf, mbuf, sem):
        pltpu.make_async_copy(emb_hbm.at[pl.ds(0, GROUP)], mbuf, sem).wait()
        pltpu.make_async_copy(emb_hbm.at[pl.ds(0, GROUP)], gbuf, sem).wait()

    def work(j, gbuf, mbuf):
        @functools.partial(plsc.parallel_loop, 0, GROUP, unroll=2)
        def _row(i):
            for k in range(H // 16):
                sl = pl.ds(k * 16, 16)
                mbuf[i, sl] = jnp.maximum(mbuf[i, sl] + gbuf[i, sl], 0.0)

        pltpu.sync_copy(mbuf, acc.at[dstbuf.at[j]], add=True)

    def chunk_step(cc, carry0):
        pltpu.sync_copy(src_hbm.at[pl.ds(wid * GPW + cc * CG, CG)], srcbuf)
        pltpu.sync_copy(dst_hbm.at[pl.ds(wid * GPW + cc * CG, CG)], dstbuf)

        start(cc, 0, gbuf0, mbuf0, sem0)

        def pair_step(p, carry):
            ja = 2 * p
            jb = 2 * p + 1
            start(cc, jb, gbuf1, mbuf1, sem1)
            drain(gbuf0, mbuf0, sem0)
            work(ja, gbuf0, mbuf0)

            @pl.when(ja + 2 < CG)
            def _():
                start(cc, ja + 2, gbuf0, mbuf0, sem0)

            drain(gbuf1, mbuf1, sem1)
            work(jb, gbuf1, mbuf1)
            return carry

        lax.fori_loop(0, CG // 2, pair_step, 0)
        return carry0

    lax.fori_loop(0, GPW // CG, chunk_step, 0)
    plsc.subcore_barrier()
    pltpu.sync_copy(acc.at[pl.ds(s * RPW, RPW)],
                    out_hbm.at[c, pl.ds(s * RPW, RPW)])


_sc_gather_scatter = functools.partial(
    pl.kernel,
    out_type=jax.ShapeDtypeStruct((NC, NPAD, H), jnp.float32),
    mesh=plsc.VectorSubcoreMesh(
        core_axis_name="c", subcore_axis_name="s",
        num_cores=NC, num_subcores=NS),
    scratch_types=[
        pltpu.VMEM((CG, GROUP), jnp.int32),
        pltpu.VMEM((CG, GROUP), jnp.int32),
        pltpu.VMEM((GROUP, H), jnp.float32),
        pltpu.VMEM((GROUP, H), jnp.float32),
        pltpu.VMEM((GROUP, H), jnp.float32),
        pltpu.VMEM((GROUP, H), jnp.float32),
        pltpu.VMEM_SHARED((NPAD, H), jnp.float32),
        pltpu.SemaphoreType.DMA,
        pltpu.SemaphoreType.DMA,
    ],
)(_sc_body)


def _enc_body(x_ref, w_ref, b_ref, o_ref):
    o_ref[...] = (
        jnp.dot(x_ref[...], w_ref[...], preferred_element_type=jnp.float32)
        + b_ref[...])


_encoder = pl.pallas_call(
    _enc_body,
    grid=(N // BN,),
    in_specs=[
        pl.BlockSpec((BN, 128), lambda i: (i, 0)),
        pl.BlockSpec((128, H), lambda i: (0, 0)),
        pl.BlockSpec((1, H), lambda i: (0, 0)),
    ],
    out_specs=pl.BlockSpec((BN, H), lambda i: (i, 0)),
    out_shape=jax.ShapeDtypeStruct((N, H), jnp.float32),
)


def _edge_body(a_ref, w_ref, b_ref, o_ref):
    o_ref[...] = (
        jnp.dot(a_ref[...], w_ref[...], preferred_element_type=jnp.float32)
        + b_ref[...])


_edge_embed = pl.pallas_call(
    _edge_body,
    grid=(EPAD // BE,),
    in_specs=[
        pl.BlockSpec((BE, D_EDGE), lambda i: (i, 0)),
        pl.BlockSpec((D_EDGE, H), lambda i: (0, 0)),
        pl.BlockSpec((1, H), lambda i: (0, 0)),
    ],
    out_specs=pl.BlockSpec((BE, H), lambda i: (i, 0)),
    out_shape=jax.ShapeDtypeStruct((EPAD, H), jnp.float32),
)


def _node_body(eps_ref, h_ref, a_ref, w1_ref, b1_ref, w2_ref, b2_ref, o_ref):
    z = h_ref[...] * eps_ref[0, 0] + a_ref[0] + a_ref[1]
    z = jnp.maximum(
        jnp.dot(z, w1_ref[...], preferred_element_type=jnp.float32)
        + b1_ref[...], 0.0)
    z = (jnp.dot(z, w2_ref[...], preferred_element_type=jnp.float32)
         + b2_ref[...])
    o_ref[...] = jnp.maximum(z, 0.0)


_node_update = pl.pallas_call(
    _node_body,
    grid=(N // BN,),
    in_specs=[
        pl.BlockSpec(memory_space=pltpu.SMEM),
        pl.BlockSpec((BN, H), lambda i: (i, 0)),
        pl.BlockSpec((NC, BN, H), lambda i: (0, i, 0)),
        pl.BlockSpec((H, H), lambda i: (0, 0)),
        pl.BlockSpec((1, H), lambda i: (0, 0)),
        pl.BlockSpec((H, H), lambda i: (0, 0)),
        pl.BlockSpec((1, H), lambda i: (0, 0)),
    ],
    out_specs=pl.BlockSpec((BN, H), lambda i: (i, 0)),
    out_shape=jax.ShapeDtypeStruct((N, H), jnp.float32),
)


def _head_body(h_ref, w0_ref, b0_ref, w1_ref, b1_ref, w2_ref, b2_ref, o_ref):
    o = jnp.maximum(
        jnp.dot(h_ref[...], w0_ref[...], preferred_element_type=jnp.float32)
        + b0_ref[...], 0.0)
    o = jnp.maximum(
        jnp.dot(o, w1_ref[...], preferred_element_type=jnp.float32)
        + b1_ref[...], 0.0)
    o_ref[...] = (
        jnp.dot(o, w2_ref[...], preferred_element_type=jnp.float32)
        + b2_ref[...])


_head = pl.pallas_call(
    _head_body,
    grid=(N // BN,),
    in_specs=[
        pl.BlockSpec((BN, H), lambda i: (i, 0)),
        pl.BlockSpec((H, H), lambda i: (0, 0)),
        pl.BlockSpec((1, H), lambda i: (0, 0)),
        pl.BlockSpec((H, H), lambda i: (0, 0)),
        pl.BlockSpec((1, H), lambda i: (0, 0)),
        pl.BlockSpec((H, H), lambda i: (0, 0)),
        pl.BlockSpec((1, H), lambda i: (0, 0)),
    ],
    out_specs=pl.BlockSpec((BN, H), lambda i: (i, 0)),
    out_shape=jax.ShapeDtypeStruct((N, H), jnp.float32),
)


def kernel(x, edge_index, edge_attr, y, params):
    p = params
    pad = EPAD - E
    src2d = jnp.concatenate(
        [edge_index[0], jnp.zeros((pad,), jnp.int32)]).reshape(EPAD // GROUP, GROUP)
    dst2d = jnp.concatenate(
        [edge_index[1], jnp.full((pad,), N, jnp.int32)]).reshape(EPAD // GROUP, GROUP)
    ea_pad = jnp.concatenate(
        [edge_attr, jnp.zeros((pad, D_EDGE), jnp.float32)], axis=0)
    zero_rows = jnp.zeros((RPW, H), jnp.float32)

    h = _encoder(x, p['enc_Wn'], p['enc_bn'].reshape(1, H))
    for l in range(L):
        wc = p['enc_We'] @ p[f'l{l}_elin_W']
        bc = p['enc_be'] @ p[f'l{l}_elin_W'] + p[f'l{l}_elin_b']
        emb = _edge_embed(ea_pad, wc, bc.reshape(1, H))
        agg2 = _sc_gather_scatter(h, emb, src2d, dst2d, zero_rows)
        g = p[f'l{l}_bn_g']
        w2 = p[f'l{l}_W2'] * g[None, :]
        b2 = p[f'l{l}_b2'] * g + p[f'l{l}_bn_b']
        epsm = (1.0 + p[f'l{l}_eps']).reshape(1, 1)
        h = _node_update(epsm, h, agg2, p[f'l{l}_W1'],
                         p[f'l{l}_b1'].reshape(1, H), w2, b2.reshape(1, H))

    w2p = jnp.pad(p['head_W2'], ((0, 0), (0, 127)))
    b2p = jnp.pad(p['head_b2'], (0, 127)).reshape(1, 128)
    o = _head(h, p['head_W0'], p['head_b0'].reshape(1, H),
              p['head_W1'], p['head_b1'].reshape(1, H), w2p, b2p)
    pred = o[:, :1]

    true_class = jnp.full((N,), -1, jnp.int32)
    true_label = jnp.where(y != -1.0, y, -1.0)
    return (pred, true_class, true_label)
